# baseline (speedup 1.0000x reference)
import functools
import numpy as np
import jax
import jax.numpy as jnp
from jax.experimental import pallas as pl
from jax.experimental.pallas import tpu as pltpu

MUL0 = 16
MUL1 = 8
D_IN = MUL0 + 3 * MUL1
EDGE_DIM = 16
HID = 64
WNUM = 576

A_EVEN = float(np.sqrt(1.0 / 24.0))
A_ODD = float(np.sqrt(3.0 / 24.0))
INV_S3 = float(1.0 / np.sqrt(3.0))

EDGE_TILE = 1000


def _msg_body(ef_ref, sh_ref, xg_ref, W1_ref, b1_ref, W2_ref, b2_ref, m_ref):
    ef = ef_ref[...]
    h = jnp.dot(ef, W1_ref[...], preferred_element_type=jnp.float32) + b1_ref[...]
    h = h * jax.nn.sigmoid(h)
    w = jnp.dot(h, W2_ref[...], preferred_element_type=jnp.float32) + b2_ref[...]

    x = xg_ref[...]
    x0 = x[:, :MUL0]
    x1f = x[:, MUL0:MUL0 + 3 * MUL1]
    sh = sh_ref[...]
    sh0 = sh[:, 0:1]
    sh1 = sh[:, 1:4]

    y0 = x0 * sh0

    dot11 = jnp.concatenate(
        [jnp.sum(x1f[:, 3 * u:3 * u + 3] * sh1, axis=1, keepdims=True)
         for u in range(MUL1)], axis=1) * INV_S3

    out0 = y0[:, 0:1] * w[:, 0:16]
    for u in range(1, MUL0):
        out0 = out0 + y0[:, u:u + 1] * w[:, 16 * u:16 * u + 16]
    for u in range(MUL1):
        out0 = out0 + dot11[:, u:u + 1] * w[:, 448 + 16 * u:448 + 16 * u + 16]
    out0 = out0 * A_EVEN

    s2 = x0[:, 0:1] * w[:, 256:264]
    for u in range(1, MUL0):
        s2 = s2 + x0[:, u:u + 1] * w[:, 256 + 8 * u:256 + 8 * u + 8]

    cols = []
    for k in range(MUL1):
        acc = x1f[:, 0:3] * w[:, 384 + k:384 + k + 1]
        for u in range(1, MUL1):
            acc = acc + x1f[:, 3 * u:3 * u + 3] * w[:, 384 + 8 * u + k:384 + 8 * u + k + 1]
        cols.append(s2[:, k:k + 1] * sh1 + sh0 * acc)
    out1 = jnp.concatenate(cols, axis=1) * (A_ODD * INV_S3)

    m_ref[...] = jnp.concatenate([out0, out1], axis=1)


def _messages(edge_feat, edge_sh, xg, W1, b1, W2, b2):
    E = edge_feat.shape[0]
    grid = (E // EDGE_TILE,)
    return pl.pallas_call(
        _msg_body,
        grid=grid,
        in_specs=[
            pl.BlockSpec((EDGE_TILE, EDGE_DIM), lambda i: (i, 0)),
            pl.BlockSpec((EDGE_TILE, 4), lambda i: (i, 0)),
            pl.BlockSpec((EDGE_TILE, D_IN), lambda i: (i, 0)),
            pl.BlockSpec((EDGE_DIM, HID), lambda i: (0, 0)),
            pl.BlockSpec((1, HID), lambda i: (0, 0)),
            pl.BlockSpec((HID, WNUM), lambda i: (0, 0)),
            pl.BlockSpec((1, WNUM), lambda i: (0, 0)),
        ],
        out_specs=pl.BlockSpec((EDGE_TILE, D_IN), lambda i: (i, 0)),
        out_shape=jax.ShapeDtypeStruct((E, D_IN), jnp.float32),
    )(edge_feat, edge_sh, xg, W1, b1.reshape(1, HID), W2, b2.reshape(1, WNUM))


def kernel(node_feats, edge_index, edge_feat, edge_sh, W1, b1, W2, b2, Wsk0, Wsk1):
    src = edge_index[0]
    dst = edge_index[1]
    xg = jnp.take(node_feats, src, axis=0)
    m = _messages(edge_feat, edge_sh, xg, W1, b1, W2, b2)
    agg = jnp.zeros((node_feats.shape[0], D_IN), jnp.float32).at[dst].add(m)
    skip0 = (node_feats[:, :MUL0] @ Wsk0) / np.sqrt(MUL0)
    skip1 = jnp.einsum('nui,uw->nwi',
                       node_feats[:, MUL0:].reshape(-1, MUL1, 3), Wsk1) / np.sqrt(MUL1)
    skip = jnp.concatenate([skip0, skip1.reshape(-1, 3 * MUL1)], axis=1)
    return skip + agg


# speedup vs baseline: 4.3174x; 4.3174x over previous
import functools
import numpy as np
import jax
import jax.numpy as jnp
from jax.experimental import pallas as pl
from jax.experimental.pallas import tpu as pltpu

MUL0 = 16
MUL1 = 8
D_IN = MUL0 + 3 * MUL1
EDGE_DIM = 16
HID = 64
WNUM = 576

A_EVEN = float(np.sqrt(1.0 / 24.0))
A_ODD = float(np.sqrt(3.0 / 24.0))
INV_S3 = float(1.0 / np.sqrt(3.0))

EDGE_TILE = 1000

_f = np.float32
REP16_16 = jnp.asarray(np.kron(np.eye(16), np.ones((1, 16))), dtype=_f)
SUM16_16 = jnp.asarray(np.kron(np.ones((16, 1)), np.eye(16)), dtype=_f)
REP8_16 = jnp.asarray(np.kron(np.eye(8), np.ones((1, 16))), dtype=_f)
SUM8_16 = jnp.asarray(np.kron(np.ones((8, 1)), np.eye(16)), dtype=_f)
REP16_8 = jnp.asarray(np.kron(np.eye(16), np.ones((1, 8))), dtype=_f)
SUM16_8 = jnp.asarray(np.kron(np.ones((16, 1)), np.eye(8)), dtype=_f)
A24_192 = jnp.asarray(np.kron(np.eye(8), np.tile(np.eye(3), (1, 8))), dtype=_f)
B64_192 = jnp.asarray(np.kron(np.eye(64), np.ones((1, 3))), dtype=_f)
C192_24 = jnp.asarray(np.kron(np.ones((8, 1)), np.eye(24)), dtype=_f)
REP8_3 = jnp.asarray(np.kron(np.eye(8), np.ones((1, 3))), dtype=_f)
_q = np.zeros((8, 24), dtype=np.float64)
_q[1:4] = np.tile(np.eye(3), (1, 8))
Q8_24 = jnp.asarray(_q, dtype=_f)
QT24_8 = jnp.asarray(np.kron(np.eye(8), np.ones((3, 1))), dtype=_f)


def _mm(a, b):
    return jnp.dot(a, b, preferred_element_type=jnp.float32)


def _msg_body(ef_ref, sh_ref, xg_ref, W1_ref, b1_ref, W2_ref, b2_ref,
              rep16_16, sum16_16, rep8_16, sum8_16, rep16_8, sum16_8,
              a24, b64, c192, rep8_3, q8, qt24, m_ref):
    T = ef_ref.shape[0]
    (rep16_16, sum16_16, rep8_16, sum8_16, rep16_8, sum16_8,
     a24, b64, c192, rep8_3, q8, qt24) = (
        r[...] for r in (rep16_16, sum16_16, rep8_16, sum8_16, rep16_8,
                         sum16_8, a24, b64, c192, rep8_3, q8, qt24))
    h = _mm(ef_ref[...], W1_ref[...]) + b1_ref[...]
    h = h * jax.nn.sigmoid(h)
    w = _mm(h, W2_ref[...]) + b2_ref[...]

    x = xg_ref[...]
    x0 = x[:, :MUL0]
    x1f = x[:, MUL0:MUL0 + 24]
    sh = sh_ref[...]
    sh0 = sh[:, 0:1]
    shp = jnp.concatenate([sh, jnp.zeros((T, 4), jnp.float32)], axis=1)
    sh1t = _mm(shp, q8)

    y0 = x0 * sh0
    o0 = _mm(_mm(y0, rep16_16) * w[:, 0:256], sum16_16)
    dot11 = _mm(x1f * sh1t, qt24) * INV_S3
    o0 = o0 + _mm(_mm(dot11, rep8_16) * w[:, 448:576], sum8_16)
    out0 = o0 * A_EVEN

    s2 = _mm(_mm(x0, rep16_8) * w[:, 256:384], sum16_8)
    m1a = _mm(s2, rep8_3) * sh1t
    p3 = _mm(x1f, a24) * _mm(w[:, 384:448], b64)
    m1b = _mm(p3, c192)
    m1 = (m1a + sh0 * m1b) * (A_ODD * INV_S3)

    m_ref[...] = jnp.concatenate([out0, m1], axis=1)


def _messages(edge_feat, edge_sh, xg, W1, b1, W2, b2):
    E = edge_feat.shape[0]
    grid = (E // EDGE_TILE,)

    def _const_spec(c):
        return pl.BlockSpec(c.shape, lambda i: (0,) * c.ndim)

    consts = [REP16_16, SUM16_16, REP8_16, SUM8_16, REP16_8, SUM16_8,
              A24_192, B64_192, C192_24, REP8_3, Q8_24, QT24_8]
    return pl.pallas_call(
        _msg_body,
        grid=grid,
        in_specs=[
            pl.BlockSpec((EDGE_TILE, EDGE_DIM), lambda i: (i, 0)),
            pl.BlockSpec((EDGE_TILE, 4), lambda i: (i, 0)),
            pl.BlockSpec((EDGE_TILE, D_IN), lambda i: (i, 0)),
            pl.BlockSpec((EDGE_DIM, HID), lambda i: (0, 0)),
            pl.BlockSpec((1, HID), lambda i: (0, 0)),
            pl.BlockSpec((HID, WNUM), lambda i: (0, 0)),
            pl.BlockSpec((1, WNUM), lambda i: (0, 0)),
        ] + [_const_spec(c) for c in consts],
        out_specs=pl.BlockSpec((EDGE_TILE, D_IN), lambda i: (i, 0)),
        out_shape=jax.ShapeDtypeStruct((E, D_IN), jnp.float32),
    )(edge_feat, edge_sh, xg, W1, b1.reshape(1, HID), W2, b2.reshape(1, WNUM),
      *consts)


def kernel(node_feats, edge_index, edge_feat, edge_sh, W1, b1, W2, b2, Wsk0, Wsk1):
    src = edge_index[0]
    dst = edge_index[1]
    xg = jnp.take(node_feats, src, axis=0)
    m = _messages(edge_feat, edge_sh, xg, W1, b1, W2, b2)
    agg = jnp.zeros((node_feats.shape[0], D_IN), jnp.float32).at[dst].add(m)
    skip0 = (node_feats[:, :MUL0] @ Wsk0) / np.sqrt(MUL0)
    skip1 = jnp.einsum('nui,uw->nwi',
                       node_feats[:, MUL0:].reshape(-1, MUL1, 3), Wsk1) / np.sqrt(MUL1)
    skip = jnp.concatenate([skip0, skip1.reshape(-1, 3 * MUL1)], axis=1)
    return skip + agg


# speedup vs baseline: 8.3914x; 1.9436x over previous
import functools
import numpy as np
import jax
import jax.numpy as jnp
from jax import lax
from jax.experimental import pallas as pl
from jax.experimental.pallas import tpu as pltpu
from jax.experimental.pallas import tpu_sc as plsc

MUL0 = 16
MUL1 = 8
D_IN = MUL0 + 3 * MUL1
DP = 48
EDGE_DIM = 16
HID = 64
WNUM = 576

A_EVEN = float(np.sqrt(1.0 / 24.0))
A_ODD = float(np.sqrt(3.0 / 24.0))
INV_S3 = float(1.0 / np.sqrt(3.0))

EDGE_TILE = 1000

NC = 2
NS = 16
NW = NC * NS
CHUNK = 128

_f = np.float32
REP16_16 = jnp.asarray(np.kron(np.eye(16), np.ones((1, 16))), dtype=_f)
SUM16_16 = jnp.asarray(np.kron(np.ones((16, 1)), np.eye(16)), dtype=_f)
REP8_16 = jnp.asarray(np.kron(np.eye(8), np.ones((1, 16))), dtype=_f)
SUM8_16 = jnp.asarray(np.kron(np.ones((8, 1)), np.eye(16)), dtype=_f)
REP16_8 = jnp.asarray(np.kron(np.eye(16), np.ones((1, 8))), dtype=_f)
SUM16_8 = jnp.asarray(np.kron(np.ones((16, 1)), np.eye(8)), dtype=_f)
A24_192 = jnp.asarray(np.kron(np.eye(8), np.tile(np.eye(3), (1, 8))), dtype=_f)
B64_192 = jnp.asarray(np.kron(np.eye(64), np.ones((1, 3))), dtype=_f)
C192_24 = jnp.asarray(np.kron(np.ones((8, 1)), np.eye(24)), dtype=_f)
REP8_3 = jnp.asarray(np.kron(np.eye(8), np.ones((1, 3))), dtype=_f)
_q = np.zeros((8, 24))
_q[1:4] = np.tile(np.eye(3), (1, 8))
Q8_24 = jnp.asarray(_q, dtype=_f)
QT24_8 = jnp.asarray(np.kron(np.eye(8), np.ones((3, 1))), dtype=_f)


def _mm(a, b):
    return jnp.dot(a, b, preferred_element_type=jnp.float32)


def _make_gather(N, E):
    per_w = E // NW
    n_full = per_w // CHUNK
    tail = per_w - n_full * CHUNK
    mesh = plsc.VectorSubcoreMesh(core_axis_name="c", subcore_axis_name="s",
                                  num_cores=NC, num_subcores=NS)

    @functools.partial(
        pl.kernel,
        out_type=jax.ShapeDtypeStruct((E, DP), jnp.float32),
        mesh=mesh,
        compiler_params=pltpu.CompilerParams(use_tc_tiling_on_sc=False),
        scratch_types=[
            pltpu.VMEM((CHUNK,), jnp.int32),
            pltpu.VMEM((CHUNK, DP), jnp.float32),
            pltpu.VMEM((tail,), jnp.int32),
            pltpu.VMEM((tail, DP), jnp.float32),
            pltpu.SemaphoreType.DMA,
        ],
    )
    def gather_k(table_hbm, idx_hbm, out_hbm, idx_v, rows_v, idx_t, rows_t, sem):
        wid = lax.axis_index("s") * NC + lax.axis_index("c")
        base = wid * per_w

        def body(i, carry):
            off = base + i * CHUNK
            pltpu.sync_copy(idx_hbm.at[pl.ds(off, CHUNK)], idx_v)
            pltpu.async_copy(table_hbm.at[idx_v], rows_v, sem).wait()
            pltpu.sync_copy(rows_v, out_hbm.at[pl.ds(off, CHUNK)])
            return carry

        lax.fori_loop(0, n_full, body, 0)
        off = base + n_full * CHUNK
        pltpu.sync_copy(idx_hbm.at[pl.ds(off, tail)], idx_t)
        pltpu.async_copy(table_hbm.at[idx_t], rows_t, sem).wait()
        pltpu.sync_copy(rows_t, out_hbm.at[pl.ds(off, tail)])

    return gather_k


def _make_scatter(N, E):
    per_w = E // NW
    n_full = per_w // CHUNK
    tail = per_w - n_full * CHUNK
    rows_per_tile = N // NS
    mesh = plsc.VectorSubcoreMesh(core_axis_name="c", subcore_axis_name="s",
                                  num_cores=NC, num_subcores=NS)

    @functools.partial(
        pl.kernel,
        out_type=jax.ShapeDtypeStruct((NC, N, DP), jnp.float32),
        mesh=mesh,
        compiler_params=pltpu.CompilerParams(use_tc_tiling_on_sc=False),
        scratch_types=[
            pltpu.VMEM((CHUNK,), jnp.int32),
            pltpu.VMEM((CHUNK, DP), jnp.float32),
            pltpu.VMEM((tail,), jnp.int32),
            pltpu.VMEM((tail, DP), jnp.float32),
            pltpu.VMEM_SHARED((N, DP), jnp.float32),
            pltpu.SemaphoreType.DMA,
        ],
    )
    def scatter_k(m_hbm, dst_hbm, zero_hbm, out_hbm,
                  idx_v, m_v, idx_t, m_t, acc_sh, sem):
        cid = lax.axis_index("c")
        sid = lax.axis_index("s")
        wid = sid * NC + cid
        base = wid * per_w
        row0 = sid * rows_per_tile

        pltpu.sync_copy(zero_hbm.at[pl.ds(row0, rows_per_tile)],
                        acc_sh.at[pl.ds(row0, rows_per_tile)])
        plsc.subcore_barrier()

        def body(i, carry):
            off = base + i * CHUNK
            pltpu.sync_copy(dst_hbm.at[pl.ds(off, CHUNK)], idx_v)
            pltpu.sync_copy(m_hbm.at[pl.ds(off, CHUNK)], m_v)
            pltpu.sync_copy(m_v, acc_sh.at[idx_v], add=True)
            return carry

        lax.fori_loop(0, n_full, body, 0)
        off = base + n_full * CHUNK
        pltpu.sync_copy(dst_hbm.at[pl.ds(off, tail)], idx_t)
        pltpu.sync_copy(m_hbm.at[pl.ds(off, tail)], m_t)
        pltpu.sync_copy(m_t, acc_sh.at[idx_t], add=True)

        plsc.subcore_barrier()
        pltpu.sync_copy(acc_sh.at[pl.ds(row0, rows_per_tile)],
                        out_hbm.at[cid, pl.ds(row0, rows_per_tile)])

    return scatter_k


def _msg_body(ef_ref, sh_ref, xg_ref, W1_ref, b1_ref, W2_ref, b2_ref,
              rep16_16, sum16_16, rep8_16, sum8_16, rep16_8, sum16_8,
              a24, b64, c192, rep8_3, q8, qt24, m_ref):
    T = ef_ref.shape[0]
    (rep16_16, sum16_16, rep8_16, sum8_16, rep16_8, sum16_8,
     a24, b64, c192, rep8_3, q8, qt24) = (
        r[...] for r in (rep16_16, sum16_16, rep8_16, sum8_16, rep16_8,
                         sum16_8, a24, b64, c192, rep8_3, q8, qt24))
    h = _mm(ef_ref[...], W1_ref[...]) + b1_ref[...]
    h = h * jax.nn.sigmoid(h)
    w = _mm(h, W2_ref[...]) + b2_ref[...]

    x = xg_ref[...]
    x0 = x[:, :MUL0]
    x1f = x[:, MUL0:MUL0 + 24]
    sh = sh_ref[...]
    sh0 = sh[:, 0:1]
    shp = jnp.concatenate([sh, jnp.zeros((T, 4), jnp.float32)], axis=1)
    sh1t = _mm(shp, q8)

    y0 = x0 * sh0
    o0 = _mm(_mm(y0, rep16_16) * w[:, 0:256], sum16_16)
    dot11 = _mm(x1f * sh1t, qt24) * INV_S3
    o0 = o0 + _mm(_mm(dot11, rep8_16) * w[:, 448:576], sum8_16)
    out0 = o0 * A_EVEN

    s2 = _mm(_mm(x0, rep16_8) * w[:, 256:384], sum16_8)
    m1a = _mm(s2, rep8_3) * sh1t
    p3 = _mm(x1f, a24) * _mm(w[:, 384:448], b64)
    m1b = _mm(p3, c192)
    m1 = (m1a + sh0 * m1b) * (A_ODD * INV_S3)

    m_ref[...] = jnp.concatenate(
        [out0, m1, jnp.zeros((T, DP - D_IN), jnp.float32)], axis=1)


def _messages(edge_feat, edge_sh, xg, W1, b1, W2, b2):
    E = edge_feat.shape[0]
    grid = (E // EDGE_TILE,)

    def _const_spec(c):
        return pl.BlockSpec(c.shape, lambda i: (0,) * c.ndim)

    consts = [REP16_16, SUM16_16, REP8_16, SUM8_16, REP16_8, SUM16_8,
              A24_192, B64_192, C192_24, REP8_3, Q8_24, QT24_8]
    return pl.pallas_call(
        _msg_body,
        grid=grid,
        in_specs=[
            pl.BlockSpec((EDGE_TILE, EDGE_DIM), lambda i: (i, 0)),
            pl.BlockSpec((EDGE_TILE, 4), lambda i: (i, 0)),
            pl.BlockSpec((EDGE_TILE, DP), lambda i: (i, 0)),
            pl.BlockSpec((EDGE_DIM, HID), lambda i: (0, 0)),
            pl.BlockSpec((1, HID), lambda i: (0, 0)),
            pl.BlockSpec((HID, WNUM), lambda i: (0, 0)),
            pl.BlockSpec((1, WNUM), lambda i: (0, 0)),
        ] + [_const_spec(c) for c in consts],
        out_specs=pl.BlockSpec((EDGE_TILE, DP), lambda i: (i, 0)),
        out_shape=jax.ShapeDtypeStruct((E, DP), jnp.float32),
    )(edge_feat, edge_sh, xg, W1, b1.reshape(1, HID), W2, b2.reshape(1, WNUM),
      *consts)


def _finish_body(nf_ref, wfull_ref, a0_ref, a1_ref, out_ref):
    out_ref[...] = (_mm(nf_ref[...], wfull_ref[...])
                    + a0_ref[...][:, :D_IN] + a1_ref[...][:, :D_IN])


def _finish(node_feats, wfull, a0, a1):
    N = node_feats.shape[0]
    TILE = 2000
    return pl.pallas_call(
        _finish_body,
        grid=(N // TILE,),
        in_specs=[
            pl.BlockSpec((TILE, D_IN), lambda i: (i, 0)),
            pl.BlockSpec((D_IN, D_IN), lambda i: (0, 0)),
            pl.BlockSpec((TILE, DP), lambda i: (i, 0)),
            pl.BlockSpec((TILE, DP), lambda i: (i, 0)),
        ],
        out_specs=pl.BlockSpec((TILE, D_IN), lambda i: (i, 0)),
        out_shape=jax.ShapeDtypeStruct((N, D_IN), jnp.float32),
    )(node_feats, wfull, a0, a1)


def kernel(node_feats, edge_index, edge_feat, edge_sh, W1, b1, W2, b2, Wsk0, Wsk1):
    N = node_feats.shape[0]
    E = edge_index.shape[1]
    src = edge_index[0]
    dst = edge_index[1]

    nf_pad = jnp.concatenate(
        [node_feats, jnp.zeros((N, DP - D_IN), jnp.float32)], axis=1)

    xg = _make_gather(N, E)(nf_pad, src)
    m = _messages(edge_feat, edge_sh, xg, W1, b1, W2, b2)
    zeros_init = jnp.zeros((N, DP), jnp.float32)
    agg = _make_scatter(N, E)(m, dst, zeros_init)

    wfull = jnp.zeros((D_IN, D_IN), jnp.float32)
    wfull = wfull.at[:MUL0, :MUL0].set(Wsk0 / np.sqrt(MUL0))
    wfull = wfull.at[MUL0:, MUL0:].set(
        jnp.kron(Wsk1, jnp.eye(3, dtype=jnp.float32)) / np.sqrt(MUL1))
    return _finish(node_feats, wfull, agg[0], agg[1])


# speedup vs baseline: 9.1989x; 1.0962x over previous
import functools
import numpy as np
import jax
import jax.numpy as jnp
from jax import lax
from jax.experimental import pallas as pl
from jax.experimental.pallas import tpu as pltpu
from jax.experimental.pallas import tpu_sc as plsc

MUL0 = 16
MUL1 = 8
D_IN = MUL0 + 3 * MUL1
DP = 48
EDGE_DIM = 16
HID = 64
WNUM = 576

A_EVEN = float(np.sqrt(1.0 / 24.0))
A_ODD = float(np.sqrt(3.0 / 24.0))
INV_S3 = float(1.0 / np.sqrt(3.0))

EDGE_TILE = 2000

NC = 2
NS = 16
NW = NC * NS
CHUNK = 128

_f = np.float32
REP16_16 = np.kron(np.eye(16), np.ones((1, 16))).astype(_f)
SUM16_16 = np.kron(np.ones((16, 1)), np.eye(16)).astype(_f)
REP8_16 = np.kron(np.eye(8), np.ones((1, 16))).astype(_f)
SUM8_16 = np.kron(np.ones((8, 1)), np.eye(16)).astype(_f)
REP16_8 = np.kron(np.eye(16), np.ones((1, 8))).astype(_f)
SUM16_8 = np.kron(np.ones((16, 1)), np.eye(8)).astype(_f)
A24_192 = np.kron(np.eye(8), np.tile(np.eye(3), (1, 8))).astype(_f)
B64_192 = np.kron(np.eye(64), np.ones((1, 3))).astype(_f)
C192_24 = np.kron(np.ones((8, 1)), np.eye(24)).astype(_f)
REP8_3 = np.kron(np.eye(8), np.ones((1, 3))).astype(_f)
_q = np.zeros((8, 24))
_q[1:4] = np.tile(np.eye(3), (1, 8))
Q8_24 = _q.astype(_f)
QT24_8 = np.kron(np.eye(8), np.ones((3, 1))).astype(_f)


def _mm(a, b):
    return jnp.dot(a, b, preferred_element_type=jnp.float32)


def _make_gather(N, E):
    per_w = E // NW
    n_full = per_w // CHUNK
    tail = per_w - n_full * CHUNK
    mesh = plsc.VectorSubcoreMesh(core_axis_name="c", subcore_axis_name="s",
                                  num_cores=NC, num_subcores=NS)

    @functools.partial(
        pl.kernel,
        out_type=jax.ShapeDtypeStruct((E, DP), jnp.float32),
        mesh=mesh,
        compiler_params=pltpu.CompilerParams(use_tc_tiling_on_sc=False),
        scratch_types=[
            pltpu.VMEM((CHUNK,), jnp.int32),
            pltpu.VMEM((CHUNK, DP), jnp.float32),
            pltpu.VMEM((tail,), jnp.int32),
            pltpu.VMEM((tail, DP), jnp.float32),
            pltpu.SemaphoreType.DMA,
        ],
    )
    def gather_k(table_hbm, idx_hbm, out_hbm, idx_v, rows_v, idx_t, rows_t, sem):
        wid = lax.axis_index("s") * NC + lax.axis_index("c")
        base = wid * per_w

        def body(i, carry):
            off = base + i * CHUNK
            pltpu.sync_copy(idx_hbm.at[pl.ds(off, CHUNK)], idx_v)
            pltpu.async_copy(table_hbm.at[idx_v], rows_v, sem).wait()
            pltpu.sync_copy(rows_v, out_hbm.at[pl.ds(off, CHUNK)])
            return carry

        lax.fori_loop(0, n_full, body, 0)
        off = base + n_full * CHUNK
        pltpu.sync_copy(idx_hbm.at[pl.ds(off, tail)], idx_t)
        pltpu.async_copy(table_hbm.at[idx_t], rows_t, sem).wait()
        pltpu.sync_copy(rows_t, out_hbm.at[pl.ds(off, tail)])

    return gather_k


def _make_scatter(N, E):
    per_w = E // NW
    n_full = per_w // CHUNK
    tail = per_w - n_full * CHUNK
    rows_per_tile = N // NS
    mesh = plsc.VectorSubcoreMesh(core_axis_name="c", subcore_axis_name="s",
                                  num_cores=NC, num_subcores=NS)

    @functools.partial(
        pl.kernel,
        out_type=jax.ShapeDtypeStruct((NC, N, DP), jnp.float32),
        mesh=mesh,
        compiler_params=pltpu.CompilerParams(use_tc_tiling_on_sc=False),
        scratch_types=[
            pltpu.VMEM((CHUNK,), jnp.int32),
            pltpu.VMEM((CHUNK, DP), jnp.float32),
            pltpu.VMEM((tail,), jnp.int32),
            pltpu.VMEM((tail, DP), jnp.float32),
            pltpu.VMEM_SHARED((N, DP), jnp.float32),
            pltpu.SemaphoreType.DMA,
        ],
    )
    def scatter_k(m_hbm, dst_hbm, zero_hbm, out_hbm,
                  idx_v, m_v, idx_t, m_t, acc_sh, sem):
        cid = lax.axis_index("c")
        sid = lax.axis_index("s")
        wid = sid * NC + cid
        base = wid * per_w
        row0 = sid * rows_per_tile

        pltpu.sync_copy(zero_hbm.at[pl.ds(row0, rows_per_tile)],
                        acc_sh.at[pl.ds(row0, rows_per_tile)])
        plsc.subcore_barrier()

        def body(i, carry):
            off = base + i * CHUNK
            pltpu.sync_copy(dst_hbm.at[pl.ds(off, CHUNK)], idx_v)
            pltpu.sync_copy(m_hbm.at[pl.ds(off, CHUNK)], m_v)
            pltpu.sync_copy(m_v, acc_sh.at[idx_v], add=True)
            return carry

        lax.fori_loop(0, n_full, body, 0)
        off = base + n_full * CHUNK
        pltpu.sync_copy(dst_hbm.at[pl.ds(off, tail)], idx_t)
        pltpu.sync_copy(m_hbm.at[pl.ds(off, tail)], m_t)
        pltpu.sync_copy(m_t, acc_sh.at[idx_t], add=True)

        plsc.subcore_barrier()
        pltpu.sync_copy(acc_sh.at[pl.ds(row0, rows_per_tile)],
                        out_hbm.at[cid, pl.ds(row0, rows_per_tile)])

    return scatter_k


def _msg_body(ef_ref, sh_ref, xg_ref, W1_ref, b1_ref, W2_ref, b2_ref,
              rep16_16, sum16_16, rep8_16, sum8_16, rep16_8, sum16_8,
              a24, b64, c192, rep8_3, q8, qt24, m_ref):
    T = ef_ref.shape[0]
    (rep16_16, sum16_16, rep8_16, sum8_16, rep16_8, sum16_8,
     a24, b64, c192, rep8_3, q8, qt24) = (
        r[...] for r in (rep16_16, sum16_16, rep8_16, sum8_16, rep16_8,
                         sum16_8, a24, b64, c192, rep8_3, q8, qt24))
    bf = jnp.bfloat16
    h = _mm(ef_ref[...].astype(bf), W1_ref[...].astype(bf)) + b1_ref[...]
    h = h * jax.nn.sigmoid(h)
    w = _mm(h.astype(bf), W2_ref[...].astype(bf)) + b2_ref[...]

    x = xg_ref[...]
    x0 = x[:, :MUL0]
    x1f = x[:, MUL0:MUL0 + 24]
    sh = sh_ref[...]
    sh0 = sh[:, 0:1]
    shp = jnp.concatenate([sh, jnp.zeros((T, 4), jnp.float32)], axis=1)
    sh1t = _mm(shp, q8)

    y0 = x0 * sh0
    o0 = _mm(_mm(y0, rep16_16) * w[:, 0:256], sum16_16)
    dot11 = _mm(x1f * sh1t, qt24) * INV_S3
    o0 = o0 + _mm(_mm(dot11, rep8_16) * w[:, 448:576], sum8_16)
    out0 = o0 * A_EVEN

    s2 = _mm(_mm(x0, rep16_8) * w[:, 256:384], sum16_8)
    m1a = _mm(s2, rep8_3) * sh1t
    p3 = _mm(x1f, a24) * _mm(w[:, 384:448], b64)
    m1b = _mm(p3, c192)
    m1 = (m1a + sh0 * m1b) * (A_ODD * INV_S3)

    m_ref[...] = jnp.concatenate(
        [out0, m1, jnp.zeros((T, DP - D_IN), jnp.float32)], axis=1)


def _messages(edge_feat, edge_sh, xg, W1, b1, W2, b2):
    E = edge_feat.shape[0]
    grid = (E // EDGE_TILE,)

    def _const_spec(c):
        return pl.BlockSpec(c.shape, lambda i: (0,) * c.ndim)

    consts = [jnp.asarray(c) for c in
              (REP16_16, SUM16_16, REP8_16, SUM8_16, REP16_8, SUM16_8,
               A24_192, B64_192, C192_24, REP8_3, Q8_24, QT24_8)]
    return pl.pallas_call(
        _msg_body,
        grid=grid,
        in_specs=[
            pl.BlockSpec((EDGE_TILE, EDGE_DIM), lambda i: (i, 0)),
            pl.BlockSpec((EDGE_TILE, 4), lambda i: (i, 0)),
            pl.BlockSpec((EDGE_TILE, DP), lambda i: (i, 0)),
            pl.BlockSpec((EDGE_DIM, HID), lambda i: (0, 0)),
            pl.BlockSpec((1, HID), lambda i: (0, 0)),
            pl.BlockSpec((HID, WNUM), lambda i: (0, 0)),
            pl.BlockSpec((1, WNUM), lambda i: (0, 0)),
        ] + [_const_spec(c) for c in consts],
        out_specs=pl.BlockSpec((EDGE_TILE, DP), lambda i: (i, 0)),
        out_shape=jax.ShapeDtypeStruct((E, DP), jnp.float32),
    )(edge_feat, edge_sh, xg, W1, b1.reshape(1, HID), W2, b2.reshape(1, WNUM),
      *consts)


def _finish_body(nf_ref, wfull_ref, a0_ref, a1_ref, out_ref):
    out_ref[...] = (_mm(nf_ref[...], wfull_ref[...])
                    + a0_ref[...][:, :D_IN] + a1_ref[...][:, :D_IN])


def _finish(node_feats, wfull, a0, a1):
    N = node_feats.shape[0]
    TILE = 2000
    return pl.pallas_call(
        _finish_body,
        grid=(N // TILE,),
        in_specs=[
            pl.BlockSpec((TILE, D_IN), lambda i: (i, 0)),
            pl.BlockSpec((D_IN, D_IN), lambda i: (0, 0)),
            pl.BlockSpec((TILE, DP), lambda i: (i, 0)),
            pl.BlockSpec((TILE, DP), lambda i: (i, 0)),
        ],
        out_specs=pl.BlockSpec((TILE, D_IN), lambda i: (i, 0)),
        out_shape=jax.ShapeDtypeStruct((N, D_IN), jnp.float32),
    )(node_feats, wfull, a0, a1)


def kernel(node_feats, edge_index, edge_feat, edge_sh, W1, b1, W2, b2, Wsk0, Wsk1):
    N = node_feats.shape[0]
    E = edge_index.shape[1]
    src = edge_index[0]
    dst = edge_index[1]

    nf_pad = jnp.concatenate(
        [node_feats, jnp.zeros((N, DP - D_IN), jnp.float32)], axis=1)

    xg = _make_gather(N, E)(nf_pad, src)
    m = _messages(edge_feat, edge_sh, xg, W1, b1, W2, b2)
    zeros_init = jnp.zeros((N, DP), jnp.float32)
    agg = _make_scatter(N, E)(m, dst, zeros_init)

    wfull = jnp.zeros((D_IN, D_IN), jnp.float32)
    wfull = wfull.at[:MUL0, :MUL0].set(Wsk0 / np.sqrt(MUL0))
    wfull = wfull.at[MUL0:, MUL0:].set(
        jnp.kron(Wsk1, jnp.eye(3, dtype=jnp.float32)) / np.sqrt(MUL1))
    return _finish(node_feats, wfull, agg[0], agg[1])


# speedup vs baseline: 9.8843x; 1.0745x over previous
import functools
import numpy as np
import jax
import jax.numpy as jnp
from jax import lax
from jax.experimental import pallas as pl
from jax.experimental.pallas import tpu as pltpu
from jax.experimental.pallas import tpu_sc as plsc

MUL0 = 16
MUL1 = 8
D_IN = MUL0 + 3 * MUL1
DP = 48
EDGE_DIM = 16
HID = 64
WNUM = 576

A_EVEN = float(np.sqrt(1.0 / 24.0))
A_ODD = float(np.sqrt(3.0 / 24.0))
INV_S3 = float(1.0 / np.sqrt(3.0))

EDGE_TILE = 4000

NC = 2
NS = 16
NW = NC * NS
CHUNK = 128

_f = np.float32
REP16_16 = np.kron(np.eye(16), np.ones((1, 16))).astype(_f)
SUM16_16 = np.kron(np.ones((16, 1)), np.eye(16)).astype(_f)
REP8_16 = np.kron(np.eye(8), np.ones((1, 16))).astype(_f)
SUM8_16 = np.kron(np.ones((8, 1)), np.eye(16)).astype(_f)
REP16_8 = np.kron(np.eye(16), np.ones((1, 8))).astype(_f)
SUM16_8 = np.kron(np.ones((16, 1)), np.eye(8)).astype(_f)
A24_192 = np.kron(np.eye(8), np.tile(np.eye(3), (1, 8))).astype(_f)
B64_192 = np.kron(np.eye(64), np.ones((1, 3))).astype(_f)
C192_24 = np.kron(np.ones((8, 1)), np.eye(24)).astype(_f)
REP8_3 = np.kron(np.eye(8), np.ones((1, 3))).astype(_f)
_q = np.zeros((8, 24))
_q[1:4] = np.tile(np.eye(3), (1, 8))
Q8_24 = _q.astype(_f)
QT24_8 = np.kron(np.eye(8), np.ones((3, 1))).astype(_f)


def _mm(a, b):
    return jnp.dot(a, b, preferred_element_type=jnp.float32)


def _mmb(a, b):
    return jnp.dot(a.astype(jnp.bfloat16), b.astype(jnp.bfloat16),
                   preferred_element_type=jnp.float32)


def _make_gather(N, E):
    per_w = E // NW
    n_full = per_w // CHUNK
    tail = per_w - n_full * CHUNK
    mesh = plsc.VectorSubcoreMesh(core_axis_name="c", subcore_axis_name="s",
                                  num_cores=NC, num_subcores=NS)

    @functools.partial(
        pl.kernel,
        out_type=jax.ShapeDtypeStruct((E, DP), jnp.float32),
        mesh=mesh,
        compiler_params=pltpu.CompilerParams(use_tc_tiling_on_sc=False),
        scratch_types=[
            pltpu.VMEM((CHUNK,), jnp.int32),
            pltpu.VMEM((CHUNK, DP), jnp.float32),
            pltpu.VMEM((tail,), jnp.int32),
            pltpu.VMEM((tail, DP), jnp.float32),
            pltpu.SemaphoreType.DMA,
        ],
    )
    def gather_k(table_hbm, idx_hbm, out_hbm, idx_v, rows_v, idx_t, rows_t, sem):
        wid = lax.axis_index("s") * NC + lax.axis_index("c")
        base = wid * per_w

        def body(i, carry):
            off = base + i * CHUNK
            pltpu.sync_copy(idx_hbm.at[pl.ds(off, CHUNK)], idx_v)
            pltpu.async_copy(table_hbm.at[idx_v], rows_v, sem).wait()
            pltpu.sync_copy(rows_v, out_hbm.at[pl.ds(off, CHUNK)])
            return carry

        lax.fori_loop(0, n_full, body, 0)
        off = base + n_full * CHUNK
        pltpu.sync_copy(idx_hbm.at[pl.ds(off, tail)], idx_t)
        pltpu.async_copy(table_hbm.at[idx_t], rows_t, sem).wait()
        pltpu.sync_copy(rows_t, out_hbm.at[pl.ds(off, tail)])

    return gather_k


def _make_scatter(N, E):
    per_w = E // NW
    n_full = per_w // CHUNK
    tail = per_w - n_full * CHUNK
    rows_per_tile = N // NS
    mesh = plsc.VectorSubcoreMesh(core_axis_name="c", subcore_axis_name="s",
                                  num_cores=NC, num_subcores=NS)

    @functools.partial(
        pl.kernel,
        out_type=jax.ShapeDtypeStruct((NC, N, DP), jnp.float32),
        mesh=mesh,
        compiler_params=pltpu.CompilerParams(use_tc_tiling_on_sc=False),
        scratch_types=[
            pltpu.VMEM((CHUNK,), jnp.int32),
            pltpu.VMEM((CHUNK, DP), jnp.float32),
            pltpu.VMEM((tail,), jnp.int32),
            pltpu.VMEM((tail, DP), jnp.float32),
            pltpu.VMEM_SHARED((N, DP), jnp.float32),
            pltpu.SemaphoreType.DMA,
        ],
    )
    def scatter_k(m_hbm, dst_hbm, zero_hbm, out_hbm,
                  idx_v, m_v, idx_t, m_t, acc_sh, sem):
        cid = lax.axis_index("c")
        sid = lax.axis_index("s")
        wid = sid * NC + cid
        base = wid * per_w
        row0 = sid * rows_per_tile

        pltpu.sync_copy(zero_hbm.at[pl.ds(row0, rows_per_tile)],
                        acc_sh.at[pl.ds(row0, rows_per_tile)])
        plsc.subcore_barrier()

        def body(i, carry):
            off = base + i * CHUNK
            pltpu.sync_copy(dst_hbm.at[pl.ds(off, CHUNK)], idx_v)
            pltpu.sync_copy(m_hbm.at[pl.ds(off, CHUNK)], m_v)
            pltpu.sync_copy(m_v, acc_sh.at[idx_v], add=True)
            return carry

        lax.fori_loop(0, n_full, body, 0)
        off = base + n_full * CHUNK
        pltpu.sync_copy(dst_hbm.at[pl.ds(off, tail)], idx_t)
        pltpu.sync_copy(m_hbm.at[pl.ds(off, tail)], m_t)
        pltpu.sync_copy(m_t, acc_sh.at[idx_t], add=True)

        plsc.subcore_barrier()
        pltpu.sync_copy(acc_sh.at[pl.ds(row0, rows_per_tile)],
                        out_hbm.at[cid, pl.ds(row0, rows_per_tile)])

    return scatter_k


def _msg_body(ef_ref, sh_ref, xg_ref, W1_ref, b1_ref, W2_ref, b2_ref,
              rep16_16, sum16_16, rep8_16, sum8_16, rep16_8, sum16_8,
              a24, b64, c192, rep8_3, q8, qt24, m_ref):
    T = ef_ref.shape[0]
    (rep16_16, sum16_16, rep8_16, sum8_16, rep16_8, sum16_8,
     a24, b64, c192, rep8_3, q8, qt24) = (
        r[...] for r in (rep16_16, sum16_16, rep8_16, sum8_16, rep16_8,
                         sum16_8, a24, b64, c192, rep8_3, q8, qt24))
    bf = jnp.bfloat16
    h = _mm(ef_ref[...].astype(bf), W1_ref[...].astype(bf)) + b1_ref[...]
    h = h * jax.nn.sigmoid(h)
    w = _mm(h.astype(bf), W2_ref[...].astype(bf)) + b2_ref[...]

    x = xg_ref[...]
    x0 = x[:, :MUL0]
    x1f = x[:, MUL0:MUL0 + 24]
    sh = sh_ref[...]
    sh0 = sh[:, 0:1]
    shp = jnp.concatenate([sh, jnp.zeros((T, 4), jnp.float32)], axis=1)
    sh1t = _mmb(shp, q8)

    y0 = x0 * sh0
    o0 = _mmb(_mmb(y0, rep16_16) * w[:, 0:256], sum16_16)
    dot11 = _mmb(x1f * sh1t, qt24) * INV_S3
    o0 = o0 + _mmb(_mmb(dot11, rep8_16) * w[:, 448:576], sum8_16)
    out0 = o0 * A_EVEN

    s2 = _mmb(_mmb(x0, rep16_8) * w[:, 256:384], sum16_8)
    m1a = _mmb(s2, rep8_3) * sh1t
    p3 = _mmb(x1f, a24) * _mmb(w[:, 384:448], b64)
    m1b = _mmb(p3, c192)
    m1 = (m1a + sh0 * m1b) * (A_ODD * INV_S3)

    m_ref[...] = jnp.concatenate(
        [out0, m1, jnp.zeros((T, DP - D_IN), jnp.float32)], axis=1)


def _messages(edge_feat, edge_sh, xg, W1, b1, W2, b2):
    E = edge_feat.shape[0]
    grid = (E // EDGE_TILE,)

    def _const_spec(c):
        return pl.BlockSpec(c.shape, lambda i: (0,) * c.ndim)

    consts = [jnp.asarray(c) for c in
              (REP16_16, SUM16_16, REP8_16, SUM8_16, REP16_8, SUM16_8,
               A24_192, B64_192, C192_24, REP8_3, Q8_24, QT24_8)]
    return pl.pallas_call(
        _msg_body,
        grid=grid,
        in_specs=[
            pl.BlockSpec((EDGE_TILE, EDGE_DIM), lambda i: (i, 0)),
            pl.BlockSpec((EDGE_TILE, 4), lambda i: (i, 0)),
            pl.BlockSpec((EDGE_TILE, DP), lambda i: (i, 0)),
            pl.BlockSpec((EDGE_DIM, HID), lambda i: (0, 0)),
            pl.BlockSpec((1, HID), lambda i: (0, 0)),
            pl.BlockSpec((HID, WNUM), lambda i: (0, 0)),
            pl.BlockSpec((1, WNUM), lambda i: (0, 0)),
        ] + [_const_spec(c) for c in consts],
        out_specs=pl.BlockSpec((EDGE_TILE, DP), lambda i: (i, 0)),
        out_shape=jax.ShapeDtypeStruct((E, DP), jnp.float32),
    )(edge_feat, edge_sh, xg, W1, b1.reshape(1, HID), W2, b2.reshape(1, WNUM),
      *consts)


def _finish_body(nf_ref, wfull_ref, a0_ref, a1_ref, out_ref):
    out_ref[...] = (_mm(nf_ref[...], wfull_ref[...])
                    + a0_ref[...][:, :D_IN] + a1_ref[...][:, :D_IN])


def _finish(node_feats, wfull, a0, a1):
    N = node_feats.shape[0]
    TILE = 2000
    return pl.pallas_call(
        _finish_body,
        grid=(N // TILE,),
        in_specs=[
            pl.BlockSpec((TILE, D_IN), lambda i: (i, 0)),
            pl.BlockSpec((D_IN, D_IN), lambda i: (0, 0)),
            pl.BlockSpec((TILE, DP), lambda i: (i, 0)),
            pl.BlockSpec((TILE, DP), lambda i: (i, 0)),
        ],
        out_specs=pl.BlockSpec((TILE, D_IN), lambda i: (i, 0)),
        out_shape=jax.ShapeDtypeStruct((N, D_IN), jnp.float32),
    )(node_feats, wfull, a0, a1)


def kernel(node_feats, edge_index, edge_feat, edge_sh, W1, b1, W2, b2, Wsk0, Wsk1):
    N = node_feats.shape[0]
    E = edge_index.shape[1]
    src = edge_index[0]
    dst = edge_index[1]

    nf_pad = jnp.concatenate(
        [node_feats, jnp.zeros((N, DP - D_IN), jnp.float32)], axis=1)

    xg = _make_gather(N, E)(nf_pad, src)
    m = _messages(edge_feat, edge_sh, xg, W1, b1, W2, b2)
    zeros_init = jnp.zeros((N, DP), jnp.float32)
    agg = _make_scatter(N, E)(m, dst, zeros_init)

    wfull = jnp.zeros((D_IN, D_IN), jnp.float32)
    wfull = wfull.at[:MUL0, :MUL0].set(Wsk0 / np.sqrt(MUL0))
    wfull = wfull.at[MUL0:, MUL0:].set(
        jnp.kron(Wsk1, jnp.eye(3, dtype=jnp.float32)) / np.sqrt(MUL1))
    return _finish(node_feats, wfull, agg[0], agg[1])


# speedup vs baseline: 9.9551x; 1.0072x over previous
import functools
import numpy as np
import jax
import jax.numpy as jnp
from jax import lax
from jax.experimental import pallas as pl
from jax.experimental.pallas import tpu as pltpu
from jax.experimental.pallas import tpu_sc as plsc

MUL0 = 16
MUL1 = 8
D_IN = MUL0 + 3 * MUL1
DP = 48
EDGE_DIM = 16
HID = 64
WNUM = 576

A_EVEN = float(np.sqrt(1.0 / 24.0))
A_ODD = float(np.sqrt(3.0 / 24.0))
INV_S3 = float(1.0 / np.sqrt(3.0))

EDGE_TILE = 8000

NC = 2
NS = 16
NW = NC * NS
CHUNK = 128

_f = np.float32
REP16_16 = np.kron(np.eye(16), np.ones((1, 16))).astype(_f)
SUM16_16 = np.kron(np.ones((16, 1)), np.eye(16)).astype(_f)
REP8_16 = np.kron(np.eye(8), np.ones((1, 16))).astype(_f)
SUM8_16 = np.kron(np.ones((8, 1)), np.eye(16)).astype(_f)
REP16_8 = np.kron(np.eye(16), np.ones((1, 8))).astype(_f)
SUM16_8 = np.kron(np.ones((16, 1)), np.eye(8)).astype(_f)
A24_192 = np.kron(np.eye(8), np.tile(np.eye(3), (1, 8))).astype(_f)
B64_192 = np.kron(np.eye(64), np.ones((1, 3))).astype(_f)
C192_24 = np.kron(np.ones((8, 1)), np.eye(24)).astype(_f)
REP8_3 = np.kron(np.eye(8), np.ones((1, 3))).astype(_f)
_q = np.zeros((8, 24))
_q[1:4] = np.tile(np.eye(3), (1, 8))
Q8_24 = _q.astype(_f)
QT24_8 = np.kron(np.eye(8), np.ones((3, 1))).astype(_f)


def _mm(a, b):
    return jnp.dot(a, b, preferred_element_type=jnp.float32)


def _mmb(a, b):
    return jnp.dot(a.astype(jnp.bfloat16), b.astype(jnp.bfloat16),
                   preferred_element_type=jnp.float32)


def _make_gather(N, E):
    per_w = E // NW
    n_full = per_w // CHUNK
    tail = per_w - n_full * CHUNK
    mesh = plsc.VectorSubcoreMesh(core_axis_name="c", subcore_axis_name="s",
                                  num_cores=NC, num_subcores=NS)

    @functools.partial(
        pl.kernel,
        out_type=jax.ShapeDtypeStruct((E, DP), jnp.float32),
        mesh=mesh,
        compiler_params=pltpu.CompilerParams(use_tc_tiling_on_sc=False),
        scratch_types=[
            pltpu.VMEM((CHUNK,), jnp.int32),
            pltpu.VMEM((CHUNK, DP), jnp.float32),
            pltpu.VMEM((tail,), jnp.int32),
            pltpu.VMEM((tail, DP), jnp.float32),
            pltpu.SemaphoreType.DMA,
        ],
    )
    def gather_k(table_hbm, idx_hbm, out_hbm, idx_v, rows_v, idx_t, rows_t, sem):
        wid = lax.axis_index("s") * NC + lax.axis_index("c")
        base = wid * per_w

        def body(i, carry):
            off = base + i * CHUNK
            pltpu.sync_copy(idx_hbm.at[pl.ds(off, CHUNK)], idx_v)
            pltpu.async_copy(table_hbm.at[idx_v], rows_v, sem).wait()
            pltpu.sync_copy(rows_v, out_hbm.at[pl.ds(off, CHUNK)])
            return carry

        lax.fori_loop(0, n_full, body, 0)
        off = base + n_full * CHUNK
        pltpu.sync_copy(idx_hbm.at[pl.ds(off, tail)], idx_t)
        pltpu.async_copy(table_hbm.at[idx_t], rows_t, sem).wait()
        pltpu.sync_copy(rows_t, out_hbm.at[pl.ds(off, tail)])

    return gather_k


def _make_scatter(N, E):
    per_w = E // NW
    n_full = per_w // CHUNK
    tail = per_w - n_full * CHUNK
    rows_per_tile = N // NS
    mesh = plsc.VectorSubcoreMesh(core_axis_name="c", subcore_axis_name="s",
                                  num_cores=NC, num_subcores=NS)

    @functools.partial(
        pl.kernel,
        out_type=jax.ShapeDtypeStruct((NC, N, DP), jnp.float32),
        mesh=mesh,
        compiler_params=pltpu.CompilerParams(use_tc_tiling_on_sc=False),
        scratch_types=[
            pltpu.VMEM((CHUNK,), jnp.int32),
            pltpu.VMEM((CHUNK, DP), jnp.float32),
            pltpu.VMEM((tail,), jnp.int32),
            pltpu.VMEM((tail, DP), jnp.float32),
            pltpu.VMEM_SHARED((N, DP), jnp.float32),
            pltpu.SemaphoreType.DMA,
        ],
    )
    def scatter_k(m_hbm, dst_hbm, zero_hbm, out_hbm,
                  idx_v, m_v, idx_t, m_t, acc_sh, sem):
        cid = lax.axis_index("c")
        sid = lax.axis_index("s")
        wid = sid * NC + cid
        base = wid * per_w
        row0 = sid * rows_per_tile

        pltpu.sync_copy(zero_hbm.at[pl.ds(row0, rows_per_tile)],
                        acc_sh.at[pl.ds(row0, rows_per_tile)])
        plsc.subcore_barrier()

        def body(i, carry):
            off = base + i * CHUNK
            pltpu.sync_copy(dst_hbm.at[pl.ds(off, CHUNK)], idx_v)
            pltpu.sync_copy(m_hbm.at[pl.ds(off, CHUNK)], m_v)
            pltpu.sync_copy(m_v, acc_sh.at[idx_v], add=True)
            return carry

        lax.fori_loop(0, n_full, body, 0)
        off = base + n_full * CHUNK
        pltpu.sync_copy(dst_hbm.at[pl.ds(off, tail)], idx_t)
        pltpu.sync_copy(m_hbm.at[pl.ds(off, tail)], m_t)
        pltpu.sync_copy(m_t, acc_sh.at[idx_t], add=True)

        plsc.subcore_barrier()
        pltpu.sync_copy(acc_sh.at[pl.ds(row0, rows_per_tile)],
                        out_hbm.at[cid, pl.ds(row0, rows_per_tile)])

    return scatter_k


def _msg_body(ef_ref, sh_ref, xg_ref, W1_ref, b1_ref, W2_ref, b2_ref,
              rep16_16, sum16_16, rep8_16, sum8_16, rep16_8, sum16_8,
              a24, b64, c192, rep8_3, q8, qt24, m_ref):
    T = ef_ref.shape[0]
    (rep16_16, sum16_16, rep8_16, sum8_16, rep16_8, sum16_8,
     a24, b64, c192, rep8_3, q8, qt24) = (
        r[...] for r in (rep16_16, sum16_16, rep8_16, sum8_16, rep16_8,
                         sum16_8, a24, b64, c192, rep8_3, q8, qt24))
    bf = jnp.bfloat16
    h = _mm(ef_ref[...].astype(bf), W1_ref[...].astype(bf)) + b1_ref[...]
    h = h * jax.nn.sigmoid(h)
    w = (_mm(h.astype(bf), W2_ref[...].astype(bf)) + b2_ref[...]).astype(bf)

    def _bb(a, b):
        return jnp.dot(a, b.astype(bf),
                       preferred_element_type=jnp.float32).astype(bf)

    def _br(a, b):
        return jnp.dot(a, b.astype(bf), preferred_element_type=jnp.float32)

    x = xg_ref[...].astype(bf)
    x0 = x[:, :MUL0]
    x1f = x[:, MUL0:MUL0 + 24]
    sh = sh_ref[...].astype(bf)
    sh0 = sh[:, 0:1]
    shp = jnp.concatenate([sh, jnp.zeros((T, 4), bf)], axis=1)
    sh1t = _bb(shp, q8)

    y0 = x0 * sh0
    o0 = _br(_bb(y0, rep16_16) * w[:, 0:256], sum16_16)
    dot11 = _bb(x1f * sh1t, qt24) * bf(INV_S3)
    o0 = o0 + _br(_bb(dot11, rep8_16) * w[:, 448:576], sum8_16)
    out0 = o0 * A_EVEN

    s2 = _br(_bb(x0, rep16_8) * w[:, 256:384], sum16_8)
    m1a = _bb(s2.astype(bf), rep8_3) * sh1t
    p3 = _bb(x1f, a24) * _bb(w[:, 384:448], b64)
    m1b = _br(p3, c192)
    m1 = (m1a.astype(jnp.float32) + sh0.astype(jnp.float32) * m1b) * (A_ODD * INV_S3)

    m_ref[...] = jnp.concatenate(
        [out0, m1, jnp.zeros((T, DP - D_IN), jnp.float32)], axis=1)


def _messages(edge_feat, edge_sh, xg, W1, b1, W2, b2):
    E = edge_feat.shape[0]
    grid = (E // EDGE_TILE,)

    def _const_spec(c):
        return pl.BlockSpec(c.shape, lambda i: (0,) * c.ndim)

    consts = [jnp.asarray(c) for c in
              (REP16_16, SUM16_16, REP8_16, SUM8_16, REP16_8, SUM16_8,
               A24_192, B64_192, C192_24, REP8_3, Q8_24, QT24_8)]
    return pl.pallas_call(
        _msg_body,
        grid=grid,
        in_specs=[
            pl.BlockSpec((EDGE_TILE, EDGE_DIM), lambda i: (i, 0)),
            pl.BlockSpec((EDGE_TILE, 4), lambda i: (i, 0)),
            pl.BlockSpec((EDGE_TILE, DP), lambda i: (i, 0)),
            pl.BlockSpec((EDGE_DIM, HID), lambda i: (0, 0)),
            pl.BlockSpec((1, HID), lambda i: (0, 0)),
            pl.BlockSpec((HID, WNUM), lambda i: (0, 0)),
            pl.BlockSpec((1, WNUM), lambda i: (0, 0)),
        ] + [_const_spec(c) for c in consts],
        out_specs=pl.BlockSpec((EDGE_TILE, DP), lambda i: (i, 0)),
        out_shape=jax.ShapeDtypeStruct((E, DP), jnp.float32),
    )(edge_feat, edge_sh, xg, W1, b1.reshape(1, HID), W2, b2.reshape(1, WNUM),
      *consts)


def _finish_body(nf_ref, wfull_ref, a0_ref, a1_ref, out_ref):
    out_ref[...] = (_mm(nf_ref[...], wfull_ref[...])
                    + a0_ref[...][:, :D_IN] + a1_ref[...][:, :D_IN])


def _finish(node_feats, wfull, a0, a1):
    N = node_feats.shape[0]
    TILE = 2000
    return pl.pallas_call(
        _finish_body,
        grid=(N // TILE,),
        in_specs=[
            pl.BlockSpec((TILE, D_IN), lambda i: (i, 0)),
            pl.BlockSpec((D_IN, D_IN), lambda i: (0, 0)),
            pl.BlockSpec((TILE, DP), lambda i: (i, 0)),
            pl.BlockSpec((TILE, DP), lambda i: (i, 0)),
        ],
        out_specs=pl.BlockSpec((TILE, D_IN), lambda i: (i, 0)),
        out_shape=jax.ShapeDtypeStruct((N, D_IN), jnp.float32),
    )(node_feats, wfull, a0, a1)


def kernel(node_feats, edge_index, edge_feat, edge_sh, W1, b1, W2, b2, Wsk0, Wsk1):
    N = node_feats.shape[0]
    E = edge_index.shape[1]
    src = edge_index[0]
    dst = edge_index[1]

    nf_pad = jnp.concatenate(
        [node_feats, jnp.zeros((N, DP - D_IN), jnp.float32)], axis=1)

    xg = _make_gather(N, E)(nf_pad, src)
    m = _messages(edge_feat, edge_sh, xg, W1, b1, W2, b2)
    zeros_init = jnp.zeros((N, DP), jnp.float32)
    agg = _make_scatter(N, E)(m, dst, zeros_init)

    wfull = jnp.zeros((D_IN, D_IN), jnp.float32)
    wfull = wfull.at[:MUL0, :MUL0].set(Wsk0 / np.sqrt(MUL0))
    wfull = wfull.at[MUL0:, MUL0:].set(
        jnp.kron(Wsk1, jnp.eye(3, dtype=jnp.float32)) / np.sqrt(MUL1))
    return _finish(node_feats, wfull, agg[0], agg[1])


# speedup vs baseline: 10.0836x; 1.0129x over previous
import functools
import numpy as np
import jax
import jax.numpy as jnp
from jax import lax
from jax.experimental import pallas as pl
from jax.experimental.pallas import tpu as pltpu
from jax.experimental.pallas import tpu_sc as plsc

MUL0 = 16
MUL1 = 8
D_IN = MUL0 + 3 * MUL1
DP = 48
EDGE_DIM = 16
HID = 64
WNUM = 576

A_EVEN = float(np.sqrt(1.0 / 24.0))
A_ODD = float(np.sqrt(3.0 / 24.0))
INV_S3 = float(1.0 / np.sqrt(3.0))

EDGE_TILE = 8000

NC = 2
NS = 16
NW = NC * NS
CHUNK = 128

_f = np.float32
REP16_16 = np.kron(np.eye(16), np.ones((1, 16))).astype(_f)
SUM16_16 = np.kron(np.ones((16, 1)), np.eye(16)).astype(_f)
REP8_16 = np.kron(np.eye(8), np.ones((1, 16))).astype(_f)
SUM8_16 = np.kron(np.ones((8, 1)), np.eye(16)).astype(_f)
REP16_8 = np.kron(np.eye(16), np.ones((1, 8))).astype(_f)
SUM16_8 = np.kron(np.ones((16, 1)), np.eye(8)).astype(_f)
A24_192 = np.kron(np.eye(8), np.tile(np.eye(3), (1, 8))).astype(_f)
B64_192 = np.kron(np.eye(64), np.ones((1, 3))).astype(_f)
C192_24 = np.kron(np.ones((8, 1)), np.eye(24)).astype(_f)
REP8_3 = np.kron(np.eye(8), np.ones((1, 3))).astype(_f)
_q = np.zeros((8, 24))
_q[1:4] = np.tile(np.eye(3), (1, 8))
Q8_24 = _q.astype(_f)
QT24_8 = np.kron(np.eye(8), np.ones((3, 1))).astype(_f)


def _mm(a, b):
    return jnp.dot(a, b, preferred_element_type=jnp.float32)


def _mmb(a, b):
    return jnp.dot(a.astype(jnp.bfloat16), b.astype(jnp.bfloat16),
                   preferred_element_type=jnp.float32)


def _make_gather(N, E):
    per_w = E // NW
    n_full = per_w // CHUNK
    tail = per_w - n_full * CHUNK
    mesh = plsc.VectorSubcoreMesh(core_axis_name="c", subcore_axis_name="s",
                                  num_cores=NC, num_subcores=NS)

    @functools.partial(
        pl.kernel,
        out_type=jax.ShapeDtypeStruct((E, DP), jnp.float32),
        mesh=mesh,
        compiler_params=pltpu.CompilerParams(use_tc_tiling_on_sc=False),
        scratch_types=[
            pltpu.VMEM((CHUNK,), jnp.int32),
            pltpu.VMEM((CHUNK, DP), jnp.float32),
            pltpu.VMEM((tail,), jnp.int32),
            pltpu.VMEM((tail, DP), jnp.float32),
            pltpu.SemaphoreType.DMA,
        ],
    )
    def gather_k(table_hbm, idx_hbm, out_hbm, idx_v, rows_v, idx_t, rows_t, sem):
        wid = lax.axis_index("s") * NC + lax.axis_index("c")
        base = wid * per_w

        def body(i, carry):
            off = base + i * CHUNK
            pltpu.sync_copy(idx_hbm.at[pl.ds(off, CHUNK)], idx_v)
            pltpu.async_copy(table_hbm.at[idx_v], rows_v, sem).wait()
            pltpu.sync_copy(rows_v, out_hbm.at[pl.ds(off, CHUNK)])
            return carry

        lax.fori_loop(0, n_full, body, 0)
        off = base + n_full * CHUNK
        pltpu.sync_copy(idx_hbm.at[pl.ds(off, tail)], idx_t)
        pltpu.async_copy(table_hbm.at[idx_t], rows_t, sem).wait()
        pltpu.sync_copy(rows_t, out_hbm.at[pl.ds(off, tail)])

    return gather_k


def _make_scatter(N, E):
    per_w = E // NW
    n_full = per_w // CHUNK
    tail = per_w - n_full * CHUNK
    rows_per_tile = N // NS
    mesh = plsc.VectorSubcoreMesh(core_axis_name="c", subcore_axis_name="s",
                                  num_cores=NC, num_subcores=NS)

    @functools.partial(
        pl.kernel,
        out_type=jax.ShapeDtypeStruct((NC, N, DP), jnp.float32),
        mesh=mesh,
        compiler_params=pltpu.CompilerParams(use_tc_tiling_on_sc=False),
        scratch_types=[
            pltpu.VMEM((CHUNK,), jnp.int32),
            pltpu.VMEM((CHUNK, DP), jnp.float32),
            pltpu.VMEM((tail,), jnp.int32),
            pltpu.VMEM((tail, DP), jnp.float32),
            pltpu.VMEM_SHARED((N, DP), jnp.float32),
            pltpu.SemaphoreType.DMA,
        ],
    )
    def scatter_k(m_hbm, dst_hbm, zero_hbm, out_hbm,
                  idx_v, m_v, idx_t, m_t, acc_sh, sem):
        cid = lax.axis_index("c")
        sid = lax.axis_index("s")
        wid = sid * NC + cid
        base = wid * per_w
        row0 = sid * rows_per_tile

        pltpu.sync_copy(zero_hbm.at[pl.ds(row0, rows_per_tile)],
                        acc_sh.at[pl.ds(row0, rows_per_tile)])
        plsc.subcore_barrier()

        def body(i, carry):
            off = base + i * CHUNK
            pltpu.sync_copy(dst_hbm.at[pl.ds(off, CHUNK)], idx_v)
            pltpu.sync_copy(m_hbm.at[pl.ds(off, CHUNK)], m_v)
            pltpu.sync_copy(m_v, acc_sh.at[idx_v], add=True)
            return carry

        lax.fori_loop(0, n_full, body, 0)
        off = base + n_full * CHUNK
        pltpu.sync_copy(dst_hbm.at[pl.ds(off, tail)], idx_t)
        pltpu.sync_copy(m_hbm.at[pl.ds(off, tail)], m_t)
        pltpu.sync_copy(m_t, acc_sh.at[idx_t], add=True)

        plsc.subcore_barrier()
        pltpu.sync_copy(acc_sh.at[pl.ds(row0, rows_per_tile)],
                        out_hbm.at[cid, pl.ds(row0, rows_per_tile)])

    return scatter_k


def _msg_body(ef_ref, sh_ref, xg_ref, W1_ref, b1_ref, W2_ref, b2_ref,
              rep16_16, sum16_16, rep8_16, sum8_16, rep16_8, sum16_8,
              a24, b64, c192, rep8_3, q8, qt24, m_ref):
    T = ef_ref.shape[0]
    (rep16_16, sum16_16, rep8_16, sum8_16, rep16_8, sum16_8,
     a24, b64, c192, rep8_3, q8, qt24) = (
        r[...] for r in (rep16_16, sum16_16, rep8_16, sum8_16, rep16_8,
                         sum16_8, a24, b64, c192, rep8_3, q8, qt24))
    bf = jnp.bfloat16
    h = _mm(ef_ref[...].astype(bf), W1_ref[...].astype(bf)) + b1_ref[...]
    h = h * jax.nn.sigmoid(h)
    w = (_mm(h.astype(bf), W2_ref[...].astype(bf)) + b2_ref[...]).astype(bf)

    def _bb(a, b):
        return jnp.dot(a, b.astype(bf),
                       preferred_element_type=jnp.float32).astype(bf)

    def _br(a, b):
        return jnp.dot(a, b.astype(bf), preferred_element_type=jnp.float32)

    x = xg_ref[...].astype(bf)
    x0 = x[:, :MUL0]
    x1f = x[:, MUL0:MUL0 + 24]
    sh = sh_ref[...].astype(bf)
    sh0 = sh[:, 0:1]
    shp = jnp.concatenate([sh, jnp.zeros((T, 4), bf)], axis=1)
    sh1t = _bb(shp, q8)

    y0 = x0 * sh0
    o0 = _br(_bb(y0, rep16_16) * w[:, 0:256], sum16_16)
    dot11 = _bb(x1f * sh1t, qt24) * bf(INV_S3)
    o0 = o0 + _br(_bb(dot11, rep8_16) * w[:, 448:576], sum8_16)
    out0 = o0 * A_EVEN

    s2 = _br(_bb(x0, rep16_8) * w[:, 256:384], sum16_8)
    m1a = _bb(s2.astype(bf), rep8_3) * sh1t
    p3 = _bb(x1f, a24) * _bb(w[:, 384:448], b64)
    m1b = _br(p3, c192)
    m1 = (m1a.astype(jnp.float32) + sh0.astype(jnp.float32) * m1b) * (A_ODD * INV_S3)

    m_ref[...] = jnp.concatenate(
        [out0, m1, jnp.zeros((T, DP - D_IN), jnp.float32)], axis=1)


def _messages(edge_feat, edge_sh, xg, W1, b1, W2, b2):
    E = edge_feat.shape[0]
    grid = (E // EDGE_TILE,)

    def _const_spec(c):
        return pl.BlockSpec(c.shape, lambda i: (0,) * c.ndim)

    consts = [jnp.asarray(c) for c in
              (REP16_16, SUM16_16, REP8_16, SUM8_16, REP16_8, SUM16_8,
               A24_192, B64_192, C192_24, REP8_3, Q8_24, QT24_8)]
    return pl.pallas_call(
        _msg_body,
        grid=grid,
        in_specs=[
            pl.BlockSpec((EDGE_TILE, EDGE_DIM), lambda i: (i, 0)),
            pl.BlockSpec((EDGE_TILE, 4), lambda i: (i, 0)),
            pl.BlockSpec((EDGE_TILE, DP), lambda i: (i, 0)),
            pl.BlockSpec((EDGE_DIM, HID), lambda i: (0, 0)),
            pl.BlockSpec((1, HID), lambda i: (0, 0)),
            pl.BlockSpec((HID, WNUM), lambda i: (0, 0)),
            pl.BlockSpec((1, WNUM), lambda i: (0, 0)),
        ] + [_const_spec(c) for c in consts],
        out_specs=pl.BlockSpec((EDGE_TILE, DP), lambda i: (i, 0)),
        out_shape=jax.ShapeDtypeStruct((E, DP), jnp.float32),
    )(edge_feat, edge_sh, xg, W1, b1.reshape(1, HID), W2, b2.reshape(1, WNUM),
      *consts)


def _finish_body(nf_ref, wfull_ref, a0_ref, a1_ref, a2_ref, a3_ref, out_ref):
    out_ref[...] = (_mm(nf_ref[...], wfull_ref[...])
                    + a0_ref[...][:, :D_IN] + a1_ref[...][:, :D_IN]
                    + a2_ref[...][:, :D_IN] + a3_ref[...][:, :D_IN])


def _finish(node_feats, wfull, a0, a1, a2, a3):
    N = node_feats.shape[0]
    TILE = 2000
    return pl.pallas_call(
        _finish_body,
        grid=(N // TILE,),
        in_specs=[
            pl.BlockSpec((TILE, D_IN), lambda i: (i, 0)),
            pl.BlockSpec((D_IN, D_IN), lambda i: (0, 0)),
            pl.BlockSpec((TILE, DP), lambda i: (i, 0)),
            pl.BlockSpec((TILE, DP), lambda i: (i, 0)),
            pl.BlockSpec((TILE, DP), lambda i: (i, 0)),
            pl.BlockSpec((TILE, DP), lambda i: (i, 0)),
        ],
        out_specs=pl.BlockSpec((TILE, D_IN), lambda i: (i, 0)),
        out_shape=jax.ShapeDtypeStruct((N, D_IN), jnp.float32),
    )(node_feats, wfull, a0, a1, a2, a3)


def kernel(node_feats, edge_index, edge_feat, edge_sh, W1, b1, W2, b2, Wsk0, Wsk1):
    N = node_feats.shape[0]
    E = edge_index.shape[1]
    src = edge_index[0]
    dst = edge_index[1]

    nf_pad = jnp.concatenate(
        [node_feats, jnp.zeros((N, DP - D_IN), jnp.float32)], axis=1)
    zeros_init = jnp.zeros((N, DP), jnp.float32)

    E1 = 96000
    parts = []
    for lo, hi in ((0, E1), (E1, E)):
        g = _make_gather(N, hi - lo)(nf_pad, src[lo:hi])
        m = _messages(edge_feat[lo:hi], edge_sh[lo:hi], g, W1, b1, W2, b2)
        parts.append(_make_scatter(N, hi - lo)(m, dst[lo:hi], zeros_init))

    wfull = jnp.zeros((D_IN, D_IN), jnp.float32)
    wfull = wfull.at[:MUL0, :MUL0].set(Wsk0 / np.sqrt(MUL0))
    wfull = wfull.at[MUL0:, MUL0:].set(
        jnp.kron(Wsk1, jnp.eye(3, dtype=jnp.float32)) / np.sqrt(MUL1))
    a, b = parts
    return _finish(node_feats, wfull, a[0], a[1], b[0], b[1])
